# Initial kernel scaffold; baseline (speedup 1.0000x reference)
import jax
import jax.numpy as jnp
from jax.experimental import pallas as pl


def kernel(node_scalar, node_vector, che_edge, che_edge_diff, che_edge_dist, che_rbf_dist, vdw_edge, vdw_edge_diff, vdw_edge_dist, vdw_rbf_dist, che_s_W1, che_s_b1, che_s_a1, che_s_W2, che_s_b2, che_s_a2, che_f_W1, che_f_b1, che_f_a1, che_f_W2, che_f_b2, che_f_a2, vdw_s_W1, vdw_s_b1, vdw_s_a1, vdw_s_W2, vdw_s_b2, vdw_s_a2, vdw_f_W1, vdw_f_b1, vdw_f_a1, vdw_f_W2, vdw_f_b2, vdw_f_a2):
    raise NotImplementedError("write your pallas kernel here")


# speedup vs baseline: 16.1576x; 16.1576x over previous
import functools

import jax
import jax.numpy as jnp
from jax import lax
from jax.experimental import pallas as pl
from jax.experimental.pallas import tpu as pltpu
from jax.experimental.pallas import tpu_sc as plsc

H = 128
CH = 128
NW = 32
BN = 1000
BE = 2000


def _prelu(x, a):
    return jnp.where(x >= 0, x, a * x)


def _node_mlp_body(ns_ref, w1c_ref, b1c_ref, w2c_ref, b2c_ref,
                   w1v_ref, b1v_ref, w2v_ref, b2v_ref, a_ref,
                   outc_ref, outv_ref):
    x = ns_ref[...]
    for (w1, b1, w2, b2, ia, out) in (
        (w1c_ref, b1c_ref, w2c_ref, b2c_ref, 0, outc_ref),
        (w1v_ref, b1v_ref, w2v_ref, b2v_ref, 2, outv_ref),
    ):
        h = lax.dot_general(x, w1[...], (((1,), (1,)), ((), ())),
                            preferred_element_type=jnp.float32)
        h = _prelu(h + b1[...], a_ref[ia])
        o = lax.dot_general(h, w2[...], (((1,), (1,)), ((), ())),
                            preferred_element_type=jnp.float32)
        out[...] = _prelu(o + b2[...], a_ref[ia + 1])


def _node_mlp(node_scalar, w1c, b1c, w2c, b2c, w1v, b1v, w2v, b2v, avec):
    n = node_scalar.shape[0]
    grid = n // BN
    full = lambda shape: pl.BlockSpec(shape, lambda i: (0, 0))
    return pl.pallas_call(
        _node_mlp_body,
        grid=(grid,),
        in_specs=[
            pl.BlockSpec((BN, H), lambda i: (i, 0)),
            full((H, H)), full((1, H)), full((3 * H, H)), full((1, 3 * H)),
            full((H, H)), full((1, H)), full((3 * H, H)), full((1, 3 * H)),
            pl.BlockSpec(memory_space=pltpu.SMEM),
        ],
        out_specs=[
            pl.BlockSpec((BN, 3 * H), lambda i: (i, 0)),
            pl.BlockSpec((BN, 3 * H), lambda i: (i, 0)),
        ],
        out_shape=[
            jax.ShapeDtypeStruct((n, 3 * H), jnp.float32),
            jax.ShapeDtypeStruct((n, 3 * H), jnp.float32),
        ],
    )(node_scalar, w1c, b1c.reshape(1, H), w2c, b2c.reshape(1, 3 * H),
      w1v, b1v.reshape(1, H), w2v, b2v.reshape(1, 3 * H), avec)


def _gather_body(nchunk, quota, tche_ref, tvdw_ref, tnv_ref, sche_ref, svdw_ref,
                 gcs_ref, gcv_ref, gvs_ref, gvv_ref,
                 idx_ref, rows_a_ref, rows_b_ref, sem_a, sem_b):
    wid = lax.axis_index("s") * 2 + lax.axis_index("c")
    start = wid * quota
    cnt = jnp.clip(nchunk - start, 0, quota)

    for src_ref, tab_a, out_a, tab_b, out_b in (
        (sche_ref, tche_ref, gcs_ref, tnv_ref, gcv_ref),
        (svdw_ref, tvdw_ref, gvs_ref, tnv_ref, gvv_ref),
    ):
        pltpu.sync_copy(src_ref.at[pl.ds(start * CH, quota * CH)], idx_ref)

        def body(j, carry):
            idx = idx_ref.at[pl.ds(j * CH, CH)]
            cp_a = pltpu.async_copy(tab_a.at[idx], rows_a_ref, sem_a)
            cp_b = pltpu.async_copy(tab_b.at[idx], rows_b_ref, sem_b)
            cp_a.wait()
            cp_b.wait()
            base = (start + j) * CH
            pltpu.sync_copy(rows_a_ref, out_a.at[pl.ds(base, CH)])
            pltpu.sync_copy(rows_b_ref, out_b.at[pl.ds(base, CH)])
            return carry

        lax.fori_loop(0, cnt, body, 0)


def _quota(nchunk):
    q = -(-nchunk // NW)
    return -(-q // 8) * 8


def _sc_gather(tab_che, tab_vdw, tab_nv, src_che_pad, src_vdw_pad, e):
    nchunk = e // CH
    quota = _quota(nchunk)
    mesh = plsc.VectorSubcoreMesh(core_axis_name="c", subcore_axis_name="s")
    out = jax.ShapeDtypeStruct((e, 3 * H), jnp.float32)
    kern = pl.kernel(
        functools.partial(_gather_body, nchunk, quota),
        out_type=[out, out, out, out],
        mesh=mesh,
        scratch_types=[
            pltpu.VMEM((quota * CH,), jnp.int32),
            pltpu.VMEM((CH, 3 * H), jnp.float32),
            pltpu.VMEM((CH, 3 * H), jnp.float32),
            pltpu.SemaphoreType.DMA,
            pltpu.SemaphoreType.DMA,
        ],
    )
    return kern(tab_che, tab_vdw, tab_nv, src_che_pad, src_vdw_pad)


def _edge_body(rbf_ref, aux_ref, gs_ref, gv_ref,
               w1_ref, b1_ref, w2_ref, b2_ref, a_ref,
               ms_ref, mv0_ref, mv1_ref, mv2_ref):
    x = rbf_ref[...]
    h = lax.dot_general(x, w1_ref[...], (((1,), (1,)), ((), ())),
                        preferred_element_type=jnp.float32)
    h = _prelu(h + b1_ref[...], a_ref[0])
    fw = lax.dot_general(h, w2_ref[...], (((1,), (1,)), ((), ())),
                         preferred_element_type=jnp.float32)
    fw = _prelu(fw + b2_ref[...], a_ref[1])
    fo = gs_ref[...] * fw
    gate = fo[:, :H]
    ge = fo[:, 2 * H:]
    aux = aux_ref[...]
    rinv = 1.0 / aux[:, 3:4]
    gv = gv_ref[...]
    ms_ref[...] = fo[:, H:2 * H]
    for k, mv in ((0, mv0_ref), (1, mv1_ref), (2, mv2_ref)):
        mv[...] = gv[:, k * H:(k + 1) * H] * gate + (aux[:, k:k + 1] * rinv) * ge


def _edge_stage(rbf_pad, aux, g_s, g_v, w1p, b1, w2, b2, avec, e):
    grid = e // BE
    full = lambda shape: pl.BlockSpec(shape, lambda i: (0, 0))
    blk = lambda w: pl.BlockSpec((BE, w), lambda i: (i, 0))
    o = jax.ShapeDtypeStruct((e, H), jnp.float32)
    return pl.pallas_call(
        _edge_body,
        grid=(grid,),
        in_specs=[
            blk(32), blk(8), blk(3 * H), blk(3 * H),
            full((H, 32)), full((1, H)), full((3 * H, H)), full((1, 3 * H)),
            pl.BlockSpec(memory_space=pltpu.SMEM),
        ],
        out_specs=[blk(H), blk(H), blk(H), blk(H)],
        out_shape=[o, o, o, o],
    )(rbf_pad, aux, g_s, g_v, w1p, b1.reshape(1, H), w2, b2.reshape(1, 3 * H),
      avec)


def _scatter_body(nchunk, n, quota,
                  mcs_ref, mc0_ref, mc1_ref, mc2_ref,
                  mvs_ref, mv0_ref, mv1_ref, mv2_ref,
                  dstc_ref, dstv_ref, zeros_ref, out_ref,
                  idx_ref, msg_ref, z_ref, w_ref, acc_ref, sem):
    cid = lax.axis_index("c")
    tid = lax.axis_index("s")
    wid = tid * 2 + cid
    start = wid * quota
    cnt = jnp.clip(nchunk - start, 0, quota)
    zrows = 80
    nzchunk = n // zrows
    ziters = -(-nzchunk // 16)

    pltpu.sync_copy(zeros_ref, z_ref)

    passes = ((mcs_ref, mvs_ref), (mc0_ref, mv0_ref),
              (mc1_ref, mv1_ref), (mc2_ref, mv2_ref))
    for p, (msg_che, msg_vdw) in enumerate(passes):
        for zi in range(ziters):
            c = zi * 16 + tid

            @pl.when(c < nzchunk)
            def _():
                pltpu.sync_copy(z_ref, acc_ref.at[pl.ds(c * zrows, zrows)])
        plsc.subcore_barrier()
        for msg, dref in ((msg_che, dstc_ref), (msg_vdw, dstv_ref)):
            def body(j, carry, msg=msg, dref=dref):
                base = (start + j) * CH
                pltpu.sync_copy(dref.at[pl.ds(base, CH)], idx_ref)
                pltpu.sync_copy(msg.at[pl.ds(base, CH)], msg_ref)
                pltpu.sync_copy(msg_ref, acc_ref.at[idx_ref], add=True)
                return carry
            lax.fori_loop(0, cnt, body, 0)
        plsc.subcore_barrier()
        obase = (p * 2 + cid) * n
        for zi in range(ziters):
            c = zi * 16 + tid

            @pl.when(c < nzchunk)
            def _():
                pltpu.sync_copy(acc_ref.at[pl.ds(c * zrows, zrows)], w_ref)
                pltpu.sync_copy(w_ref, out_ref.at[pl.ds(obase + c * zrows, zrows)])
        plsc.subcore_barrier()


def _sc_scatter(msgs_che, msgs_vdw, dst_che, dst_vdw, zeros, e, n):
    nchunk = e // CH
    quota = _quota(nchunk)
    mesh = plsc.VectorSubcoreMesh(core_axis_name="c", subcore_axis_name="s")
    kern = pl.kernel(
        functools.partial(_scatter_body, nchunk, n, quota),
        out_type=jax.ShapeDtypeStruct((8 * n, H), jnp.float32),
        mesh=mesh,
        scratch_types=[
            pltpu.VMEM((CH,), jnp.int32),
            pltpu.VMEM((CH, H), jnp.float32),
            pltpu.VMEM((80, H), jnp.float32),
            pltpu.VMEM((80, H), jnp.float32),
            pltpu.VMEM_SHARED((n, H), jnp.float32),
            pltpu.SemaphoreType.DMA,
        ],
    )
    return kern(*msgs_che, *msgs_vdw, dst_che, dst_vdw, zeros)


def _combine_body(ns_ref, nv0_ref, nv1_ref, nv2_ref,
                  ps0_ref, ps1_ref, p00_ref, p01_ref,
                  p10_ref, p11_ref, p20_ref, p21_ref,
                  os_ref, ov0_ref, ov1_ref, ov2_ref):
    os_ref[...] = ns_ref[...] + ps0_ref[...] + ps1_ref[...]
    ov0_ref[...] = nv0_ref[...] + p00_ref[...] + p01_ref[...]
    ov1_ref[...] = nv1_ref[...] + p10_ref[...] + p11_ref[...]
    ov2_ref[...] = nv2_ref[...] + p20_ref[...] + p21_ref[...]


def _combine(node_scalar, nv0, nv1, nv2, part, n):
    grid = n // BN
    nb = n // BN
    blk = pl.BlockSpec((BN, H), lambda i: (i, 0))

    def pblk(p, c):
        off = (p * 2 + c) * nb
        return pl.BlockSpec((BN, H), lambda i, off=off: (off + i, 0))

    o = jax.ShapeDtypeStruct((n, H), jnp.float32)
    return pl.pallas_call(
        _combine_body,
        grid=(grid,),
        in_specs=[blk, blk, blk, blk,
                  pblk(0, 0), pblk(0, 1), pblk(1, 0), pblk(1, 1),
                  pblk(2, 0), pblk(2, 1), pblk(3, 0), pblk(3, 1)],
        out_specs=[blk, blk, blk, blk],
        out_shape=[o, o, o, o],
    )(node_scalar, nv0, nv1, nv2,
      part, part, part, part, part, part, part, part)


def kernel(node_scalar, node_vector, che_edge, che_edge_diff, che_edge_dist,
           che_rbf_dist, vdw_edge, vdw_edge_diff, vdw_edge_dist, vdw_rbf_dist,
           che_s_W1, che_s_b1, che_s_a1, che_s_W2, che_s_b2, che_s_a2,
           che_f_W1, che_f_b1, che_f_a1, che_f_W2, che_f_b2, che_f_a2,
           vdw_s_W1, vdw_s_b1, vdw_s_a1, vdw_s_W2, vdw_s_b2, vdw_s_a2,
           vdw_f_W1, vdw_f_b1, vdw_f_a1, vdw_f_W2, vdw_f_b2, vdw_f_a2):
    n = node_scalar.shape[0]
    e = che_edge.shape[0]
    nchunk = e // CH
    pad_e = NW * _quota(nchunk) * CH

    a_s = jnp.stack([che_s_a1, che_s_a2, vdw_s_a1, vdw_s_a2])
    tab_che, tab_vdw = _node_mlp(node_scalar, che_s_W1, che_s_b1, che_s_W2,
                                 che_s_b2, vdw_s_W1, vdw_s_b1, vdw_s_W2,
                                 vdw_s_b2, a_s)
    tab_nv = node_vector.reshape(n, 3 * H)

    src_che = jnp.pad(che_edge[:, 1], (0, pad_e - e))
    src_vdw = jnp.pad(vdw_edge[:, 1], (0, pad_e - e))
    g_che_s, g_che_v, g_vdw_s, g_vdw_v = _sc_gather(
        tab_che, tab_vdw, tab_nv, src_che, src_vdw, e)

    a_fc = jnp.stack([che_f_a1, che_f_a2])
    a_fv = jnp.stack([vdw_f_a1, vdw_f_a2])
    ee = che_rbf_dist.shape[1]
    rbf_che = jnp.pad(che_rbf_dist, ((0, 0), (0, 32 - ee)))
    rbf_vdw = jnp.pad(vdw_rbf_dist, ((0, 0), (0, 32 - ee)))
    w1c = jnp.pad(che_f_W1, ((0, 0), (0, 32 - ee)))
    w1v = jnp.pad(vdw_f_W1, ((0, 0), (0, 32 - ee)))
    aux_che = jnp.pad(
        jnp.concatenate([che_edge_diff, che_edge_dist[:, None]], axis=1),
        ((0, 0), (0, 4)))
    aux_vdw = jnp.pad(
        jnp.concatenate([vdw_edge_diff, vdw_edge_dist[:, None]], axis=1),
        ((0, 0), (0, 4)))
    msgs_che = _edge_stage(rbf_che, aux_che, g_che_s, g_che_v,
                           w1c, che_f_b1, che_f_W2, che_f_b2, a_fc, e)
    msgs_vdw = _edge_stage(rbf_vdw, aux_vdw, g_vdw_s, g_vdw_v,
                           w1v, vdw_f_b1, vdw_f_W2, vdw_f_b2, a_fv, e)

    dst_che = jnp.pad(che_edge[:, 0], (0, pad_e - e))
    dst_vdw = jnp.pad(vdw_edge[:, 0], (0, pad_e - e))
    zeros = jnp.zeros((80, H), jnp.float32)
    part = _sc_scatter(msgs_che, msgs_vdw, dst_che, dst_vdw, zeros, e, n)

    nv0 = node_vector[:, 0, :]
    nv1 = node_vector[:, 1, :]
    nv2 = node_vector[:, 2, :]
    os_, ov0, ov1, ov2 = _combine(node_scalar, nv0, nv1, nv2, part, n)
    return os_, jnp.stack([ov0, ov1, ov2], axis=1)


# speedup vs baseline: 19.8512x; 1.2286x over previous
import functools

import jax
import jax.numpy as jnp
from jax import lax
from jax.experimental import pallas as pl
from jax.experimental.pallas import tpu as pltpu
from jax.experimental.pallas import tpu_sc as plsc

H = 128
CH = 128
NW = 32
BN = 1000
BE = 2000


def _prelu(x, a):
    return jnp.where(x >= 0, x, a * x)


def _node_mlp_body(ns_ref, nv_ref, w1c_ref, b1c_ref, w2c_ref, b2c_ref,
                   w1v_ref, b1v_ref, w2v_ref, b2v_ref, a_ref,
                   outc_ref, outv_ref):
    x = ns_ref[...]
    nv = nv_ref[...]
    for (w1, b1, w2, b2, ia, out) in (
        (w1c_ref, b1c_ref, w2c_ref, b2c_ref, 0, outc_ref),
        (w1v_ref, b1v_ref, w2v_ref, b2v_ref, 2, outv_ref),
    ):
        h = lax.dot_general(x, w1[...], (((1,), (1,)), ((), ())),
                            preferred_element_type=jnp.float32)
        h = _prelu(h + b1[...], a_ref[ia])
        o = lax.dot_general(h, w2[...], (((1,), (1,)), ((), ())),
                            preferred_element_type=jnp.float32)
        out[:, :3 * H] = _prelu(o + b2[...], a_ref[ia + 1])
        out[:, 3 * H:] = nv


def _node_mlp(node_scalar, tab_nv, w1c, b1c, w2c, b2c, w1v, b1v, w2v, b2v,
              avec):
    n = node_scalar.shape[0]
    grid = n // BN
    full = lambda shape: pl.BlockSpec(shape, lambda i: (0, 0))
    return pl.pallas_call(
        _node_mlp_body,
        grid=(grid,),
        in_specs=[
            pl.BlockSpec((BN, H), lambda i: (i, 0)),
            pl.BlockSpec((BN, 3 * H), lambda i: (i, 0)),
            full((H, H)), full((1, H)), full((3 * H, H)), full((1, 3 * H)),
            full((H, H)), full((1, H)), full((3 * H, H)), full((1, 3 * H)),
            pl.BlockSpec(memory_space=pltpu.SMEM),
        ],
        out_specs=[
            pl.BlockSpec((BN, 6 * H), lambda i: (i, 0)),
            pl.BlockSpec((BN, 6 * H), lambda i: (i, 0)),
        ],
        out_shape=[
            jax.ShapeDtypeStruct((n, 6 * H), jnp.float32),
            jax.ShapeDtypeStruct((n, 6 * H), jnp.float32),
        ],
    )(node_scalar, tab_nv, w1c, b1c.reshape(1, H), w2c, b2c.reshape(1, 3 * H),
      w1v, b1v.reshape(1, H), w2v, b2v.reshape(1, 3 * H), avec)


GCH = 64


def _gather_body(nchunk, quota, tab_ref, src_ref, g_ref,
                 idx_ref, buf0_ref, buf1_ref, sem0, sem1):
    wid = lax.axis_index("s") * 2 + lax.axis_index("c")
    start = wid * quota
    cnt = jnp.clip(nchunk - start, 0, quota)
    bufs = (buf0_ref, buf1_ref)
    sems = (sem0, sem1)

    pltpu.sync_copy(src_ref.at[pl.ds(start * GCH, quota * GCH)], idx_ref)

    def _start(j, b):
        idx = idx_ref.at[pl.ds(j * GCH, GCH)]
        pltpu.async_copy(tab_ref.at[idx], bufs[b], sems[b])

    for b in (0, 1):
        @pl.when(b < cnt)
        def _(b=b):
            _start(b, b)

    def body(g, carry):
        for b in (0, 1):
            j = g * 2 + b

            @pl.when(j < cnt)
            def _(j=j, b=b):
                pltpu.make_async_copy(tab_ref.at[idx_ref.at[pl.ds(0, GCH)]],
                                      bufs[b], sems[b]).wait()
                pltpu.sync_copy(bufs[b], g_ref.at[pl.ds((start + j) * GCH, GCH)])

            @pl.when(j + 2 < cnt)
            def _(j=j, b=b):
                _start(j + 2, b)
        return carry

    lax.fori_loop(0, quota // 2, body, 0)


def _quota(nchunk):
    q = -(-nchunk // NW)
    return -(-q // 8) * 8


def _sc_gather(tab, src_pad, e):
    nchunk = e // GCH
    quota = _quota(nchunk)
    mesh = plsc.VectorSubcoreMesh(core_axis_name="c", subcore_axis_name="s")
    kern = pl.kernel(
        functools.partial(_gather_body, nchunk, quota),
        out_type=jax.ShapeDtypeStruct((e, 6 * H), jnp.float32),
        mesh=mesh,
        scratch_types=[
            pltpu.VMEM((quota * GCH,), jnp.int32),
            pltpu.VMEM((GCH, 6 * H), jnp.float32),
            pltpu.VMEM((GCH, 6 * H), jnp.float32),
            pltpu.SemaphoreType.DMA,
            pltpu.SemaphoreType.DMA,
        ],
    )
    return kern(tab, src_pad)


def _edge_body(rbf_ref, aux_ref, g_ref,
               w1_ref, b1_ref, w2_ref, b2_ref, a_ref,
               ms_ref, mv0_ref, mv1_ref, mv2_ref):
    x = rbf_ref[...]
    h = lax.dot_general(x, w1_ref[...], (((1,), (1,)), ((), ())),
                        preferred_element_type=jnp.float32)
    h = _prelu(h + b1_ref[...], a_ref[0])
    fw = lax.dot_general(h, w2_ref[...], (((1,), (1,)), ((), ())),
                         preferred_element_type=jnp.float32)
    fw = _prelu(fw + b2_ref[...], a_ref[1])
    g = g_ref[...]
    fo = g[:, :3 * H] * fw
    gate = fo[:, :H]
    ge = fo[:, 2 * H:]
    aux = aux_ref[...]
    rinv = 1.0 / aux[:, 3:4]
    ms_ref[...] = fo[:, H:2 * H]
    for k, mv in ((0, mv0_ref), (1, mv1_ref), (2, mv2_ref)):
        mv[...] = (g[:, (3 + k) * H:(4 + k) * H] * gate
                   + (aux[:, k:k + 1] * rinv) * ge)


def _edge_stage(rbf_pad, aux, g, w1p, b1, w2, b2, avec, e):
    grid = e // BE
    full = lambda shape: pl.BlockSpec(shape, lambda i: (0, 0))
    blk = lambda w: pl.BlockSpec((BE, w), lambda i: (i, 0))
    o = jax.ShapeDtypeStruct((e, H), jnp.float32)
    return pl.pallas_call(
        _edge_body,
        grid=(grid,),
        in_specs=[
            blk(32), blk(8), blk(6 * H),
            full((H, 32)), full((1, H)), full((3 * H, H)), full((1, 3 * H)),
            pl.BlockSpec(memory_space=pltpu.SMEM),
        ],
        out_specs=[blk(H), blk(H), blk(H), blk(H)],
        out_shape=[o, o, o, o],
    )(rbf_pad, aux, g, w1p, b1.reshape(1, H), w2, b2.reshape(1, 3 * H), avec)


def _scatter_body(nchunk, n, quota,
                  mcs_ref, mc0_ref, mc1_ref, mc2_ref,
                  mvs_ref, mv0_ref, mv1_ref, mv2_ref,
                  dstc_ref, dstv_ref, zeros_ref, out_ref,
                  idx0_ref, idx1_ref, msg0_ref, msg1_ref, w_ref,
                  acc_ref, msem0, msem1, isem0, isem1):
    cid = lax.axis_index("c")
    tid = lax.axis_index("s")
    wid = tid * 2 + cid
    start = wid * quota
    cnt = jnp.clip(nchunk - start, 0, quota)
    zrows = 80
    nzchunk = n // zrows
    ziters = -(-nzchunk // 16)

    mbufs = (msg0_ref, msg1_ref)
    ibufs = (idx0_ref, idx1_ref)
    msems = (msem0, msem1)
    isems = (isem0, isem1)

    passes = ((mcs_ref, mvs_ref), (mc0_ref, mv0_ref),
              (mc1_ref, mv1_ref), (mc2_ref, mv2_ref))
    for p, (msg_che, msg_vdw) in enumerate(passes):
        pltpu.sync_copy(zeros_ref, w_ref)
        for zi in range(ziters):
            c = zi * 16 + tid

            @pl.when(c < nzchunk)
            def _():
                pltpu.sync_copy(w_ref, acc_ref.at[pl.ds(c * zrows, zrows)])
        plsc.subcore_barrier()
        for msg, dref in ((msg_che, dstc_ref), (msg_vdw, dstv_ref)):
            def _start(j, b, msg=msg, dref=dref):
                base = (start + j) * CH
                pltpu.async_copy(msg.at[pl.ds(base, CH)], mbufs[b], msems[b])
                pltpu.async_copy(dref.at[pl.ds(base, CH)], ibufs[b], isems[b])

            for b in (0, 1):
                @pl.when(b < cnt)
                def _(b=b):
                    _start(b, b)

            def body(g, carry, msg=msg, dref=dref, _start=_start):
                for b in (0, 1):
                    j = g * 2 + b

                    @pl.when(j < cnt)
                    def _(j=j, b=b):
                        pltpu.make_async_copy(
                            msg.at[pl.ds(0, CH)], mbufs[b], msems[b]).wait()
                        pltpu.make_async_copy(
                            dref.at[pl.ds(0, CH)], ibufs[b], isems[b]).wait()
                        pltpu.sync_copy(mbufs[b], acc_ref.at[ibufs[b]],
                                        add=True)

                    @pl.when(j + 2 < cnt)
                    def _(j=j, b=b):
                        _start(j + 2, b)
                return carry

            lax.fori_loop(0, quota // 2, body, 0)
        plsc.subcore_barrier()
        obase = (p * 2 + cid) * n
        for zi in range(ziters):
            c = zi * 16 + tid

            @pl.when(c < nzchunk)
            def _():
                pltpu.sync_copy(acc_ref.at[pl.ds(c * zrows, zrows)], w_ref)
                pltpu.sync_copy(w_ref, out_ref.at[pl.ds(obase + c * zrows, zrows)])
        plsc.subcore_barrier()


def _sc_scatter(msgs_che, msgs_vdw, dst_che, dst_vdw, zeros, e, n):
    nchunk = e // CH
    quota = _quota(nchunk)
    mesh = plsc.VectorSubcoreMesh(core_axis_name="c", subcore_axis_name="s")
    kern = pl.kernel(
        functools.partial(_scatter_body, nchunk, n, quota),
        out_type=jax.ShapeDtypeStruct((8 * n, H), jnp.float32),
        mesh=mesh,
        scratch_types=[
            pltpu.VMEM((CH,), jnp.int32),
            pltpu.VMEM((CH,), jnp.int32),
            pltpu.VMEM((CH, H), jnp.float32),
            pltpu.VMEM((CH, H), jnp.float32),
            pltpu.VMEM((80, H), jnp.float32),
            pltpu.VMEM_SHARED((n, H), jnp.float32),
            pltpu.SemaphoreType.DMA,
            pltpu.SemaphoreType.DMA,
            pltpu.SemaphoreType.DMA,
            pltpu.SemaphoreType.DMA,
        ],
    )
    return kern(*msgs_che, *msgs_vdw, dst_che, dst_vdw, zeros)


def _combine_body(ns_ref, nv0_ref, nv1_ref, nv2_ref,
                  ps0_ref, ps1_ref, p00_ref, p01_ref,
                  p10_ref, p11_ref, p20_ref, p21_ref,
                  os_ref, ov0_ref, ov1_ref, ov2_ref):
    os_ref[...] = ns_ref[...] + ps0_ref[...] + ps1_ref[...]
    ov0_ref[...] = nv0_ref[...] + p00_ref[...] + p01_ref[...]
    ov1_ref[...] = nv1_ref[...] + p10_ref[...] + p11_ref[...]
    ov2_ref[...] = nv2_ref[...] + p20_ref[...] + p21_ref[...]


def _combine(node_scalar, nv0, nv1, nv2, part, n):
    grid = n // BN
    nb = n // BN
    blk = pl.BlockSpec((BN, H), lambda i: (i, 0))

    def pblk(p, c):
        off = (p * 2 + c) * nb
        return pl.BlockSpec((BN, H), lambda i, off=off: (off + i, 0))

    o = jax.ShapeDtypeStruct((n, H), jnp.float32)
    return pl.pallas_call(
        _combine_body,
        grid=(grid,),
        in_specs=[blk, blk, blk, blk,
                  pblk(0, 0), pblk(0, 1), pblk(1, 0), pblk(1, 1),
                  pblk(2, 0), pblk(2, 1), pblk(3, 0), pblk(3, 1)],
        out_specs=[blk, blk, blk, blk],
        out_shape=[o, o, o, o],
    )(node_scalar, nv0, nv1, nv2,
      part, part, part, part, part, part, part, part)


def kernel(node_scalar, node_vector, che_edge, che_edge_diff, che_edge_dist,
           che_rbf_dist, vdw_edge, vdw_edge_diff, vdw_edge_dist, vdw_rbf_dist,
           che_s_W1, che_s_b1, che_s_a1, che_s_W2, che_s_b2, che_s_a2,
           che_f_W1, che_f_b1, che_f_a1, che_f_W2, che_f_b2, che_f_a2,
           vdw_s_W1, vdw_s_b1, vdw_s_a1, vdw_s_W2, vdw_s_b2, vdw_s_a2,
           vdw_f_W1, vdw_f_b1, vdw_f_a1, vdw_f_W2, vdw_f_b2, vdw_f_a2):
    n = node_scalar.shape[0]
    e = che_edge.shape[0]
    pad_e = NW * _quota(e // CH) * CH
    pad_g = NW * _quota(e // GCH) * GCH

    a_s = jnp.stack([che_s_a1, che_s_a2, vdw_s_a1, vdw_s_a2])
    tab_nv = node_vector.reshape(n, 3 * H)
    tab_che, tab_vdw = _node_mlp(node_scalar, tab_nv, che_s_W1, che_s_b1,
                                 che_s_W2, che_s_b2, vdw_s_W1, vdw_s_b1,
                                 vdw_s_W2, vdw_s_b2, a_s)

    src_che = jnp.pad(che_edge[:, 1], (0, pad_g - e))
    src_vdw = jnp.pad(vdw_edge[:, 1], (0, pad_g - e))
    g_che = _sc_gather(tab_che, src_che, e)
    g_vdw = _sc_gather(tab_vdw, src_vdw, e)

    a_fc = jnp.stack([che_f_a1, che_f_a2])
    a_fv = jnp.stack([vdw_f_a1, vdw_f_a2])
    ee = che_rbf_dist.shape[1]
    rbf_che = jnp.pad(che_rbf_dist, ((0, 0), (0, 32 - ee)))
    rbf_vdw = jnp.pad(vdw_rbf_dist, ((0, 0), (0, 32 - ee)))
    w1c = jnp.pad(che_f_W1, ((0, 0), (0, 32 - ee)))
    w1v = jnp.pad(vdw_f_W1, ((0, 0), (0, 32 - ee)))
    aux_che = jnp.pad(
        jnp.concatenate([che_edge_diff, che_edge_dist[:, None]], axis=1),
        ((0, 0), (0, 4)))
    aux_vdw = jnp.pad(
        jnp.concatenate([vdw_edge_diff, vdw_edge_dist[:, None]], axis=1),
        ((0, 0), (0, 4)))
    msgs_che = _edge_stage(rbf_che, aux_che, g_che,
                           w1c, che_f_b1, che_f_W2, che_f_b2, a_fc, e)
    msgs_vdw = _edge_stage(rbf_vdw, aux_vdw, g_vdw,
                           w1v, vdw_f_b1, vdw_f_W2, vdw_f_b2, a_fv, e)

    dst_che = jnp.pad(che_edge[:, 0], (0, pad_e - e))
    dst_vdw = jnp.pad(vdw_edge[:, 0], (0, pad_e - e))
    zeros = jnp.zeros((80, H), jnp.float32)
    part = _sc_scatter(msgs_che, msgs_vdw, dst_che, dst_vdw, zeros, e, n)

    nv0 = node_vector[:, 0, :]
    nv1 = node_vector[:, 1, :]
    nv2 = node_vector[:, 2, :]
    os_, ov0, ov1, ov2 = _combine(node_scalar, nv0, nv1, nv2, part, n)
    return os_, jnp.stack([ov0, ov1, ov2], axis=1)


# speedup vs baseline: 24.4606x; 1.2322x over previous
import functools

import jax
import jax.numpy as jnp
from jax import lax
from jax.experimental import pallas as pl
from jax.experimental.pallas import tpu as pltpu
from jax.experimental.pallas import tpu_sc as plsc

H = 128
CH = 128
NW = 32
BN = 1000
BE = 2000


def _prelu(x, a):
    return jnp.where(x >= 0, x, a * x)


def _node_mlp_body(ns_ref, nv_ref, w1c_ref, b1c_ref, w2c_ref, b2c_ref,
                   w1v_ref, b1v_ref, w2v_ref, b2v_ref, a_ref,
                   outc_ref, outv_ref):
    x = ns_ref[...]
    nv = nv_ref[...]
    for (w1, b1, w2, b2, ia, out) in (
        (w1c_ref, b1c_ref, w2c_ref, b2c_ref, 0, outc_ref),
        (w1v_ref, b1v_ref, w2v_ref, b2v_ref, 2, outv_ref),
    ):
        h = lax.dot_general(x, w1[...], (((1,), (1,)), ((), ())),
                            preferred_element_type=jnp.float32)
        h = _prelu(h + b1[...], a_ref[ia])
        o = lax.dot_general(h, w2[...], (((1,), (1,)), ((), ())),
                            preferred_element_type=jnp.float32)
        out[:, :3 * H] = _prelu(o + b2[...], a_ref[ia + 1])
        out[:, 3 * H:] = nv


def _node_mlp(node_scalar, tab_nv, w1c, b1c, w2c, b2c, w1v, b1v, w2v, b2v,
              avec):
    n = node_scalar.shape[0]
    grid = n // BN
    full = lambda shape: pl.BlockSpec(shape, lambda i: (0, 0))
    return pl.pallas_call(
        _node_mlp_body,
        grid=(grid,),
        in_specs=[
            pl.BlockSpec((BN, H), lambda i: (i, 0)),
            pl.BlockSpec((BN, 3 * H), lambda i: (i, 0)),
            full((H, H)), full((1, H)), full((3 * H, H)), full((1, 3 * H)),
            full((H, H)), full((1, H)), full((3 * H, H)), full((1, 3 * H)),
            pl.BlockSpec(memory_space=pltpu.SMEM),
        ],
        out_specs=[
            pl.BlockSpec((BN, 6 * H), lambda i: (i, 0)),
            pl.BlockSpec((BN, 6 * H), lambda i: (i, 0)),
        ],
        out_shape=[
            jax.ShapeDtypeStruct((n, 6 * H), jnp.float32),
            jax.ShapeDtypeStruct((n, 6 * H), jnp.float32),
        ],
    )(node_scalar, tab_nv, w1c, b1c.reshape(1, H), w2c, b2c.reshape(1, 3 * H),
      w1v, b1v.reshape(1, H), w2v, b2v.reshape(1, 3 * H), avec)


GCH = 128


def _gather_body(nchunk, quota, tab_ref, src_ref, g_ref,
                 idx_ref, buf0_ref, buf1_ref, sem0, sem1):
    wid = lax.axis_index("s") * 2 + lax.axis_index("c")
    start = wid * quota
    cnt = jnp.clip(nchunk - start, 0, quota)
    bufs = (buf0_ref, buf1_ref)
    sems = (sem0, sem1)

    pltpu.sync_copy(src_ref.at[pl.ds(start * GCH, quota * GCH)], idx_ref)

    def _start(j, b):
        idx = idx_ref.at[pl.ds(j * GCH, GCH)]
        pltpu.async_copy(tab_ref.at[idx], bufs[b], sems[b])

    for b in (0, 1):
        @pl.when(b < cnt)
        def _(b=b):
            _start(b, b)

    def body(g, carry):
        for b in (0, 1):
            j = g * 2 + b

            @pl.when(j < cnt)
            def _(j=j, b=b):
                pltpu.make_async_copy(tab_ref.at[idx_ref.at[pl.ds(0, GCH)]],
                                      bufs[b], sems[b]).wait()
                pltpu.sync_copy(bufs[b], g_ref.at[pl.ds((start + j) * GCH, GCH)])

            @pl.when(j + 2 < cnt)
            def _(j=j, b=b):
                _start(j + 2, b)
        return carry

    lax.fori_loop(0, quota // 2, body, 0)


def _quota(nchunk):
    q = -(-nchunk // NW)
    return -(-q // 8) * 8


def _sc_gather(tab, src_pad, e):
    nchunk = e // GCH
    quota = _quota(nchunk)
    mesh = plsc.VectorSubcoreMesh(core_axis_name="c", subcore_axis_name="s")
    kern = pl.kernel(
        functools.partial(_gather_body, nchunk, quota),
        out_type=jax.ShapeDtypeStruct((e, 3 * H), jnp.float32),
        mesh=mesh,
        scratch_types=[
            pltpu.VMEM((quota * GCH,), jnp.int32),
            pltpu.VMEM((GCH, 3 * H), jnp.float32),
            pltpu.VMEM((GCH, 3 * H), jnp.float32),
            pltpu.SemaphoreType.DMA,
            pltpu.SemaphoreType.DMA,
        ],
    )
    return kern(tab, src_pad)


def _edge_body(rbf_ref, aux_ref, g_ref,
               w1_ref, b1_ref, w2_ref, b2_ref, a_ref,
               ms_ref, mv0_ref, mv1_ref, mv2_ref):
    x = rbf_ref[...]
    h = lax.dot_general(x, w1_ref[...], (((1,), (1,)), ((), ())),
                        preferred_element_type=jnp.float32)
    h = _prelu(h + b1_ref[...], a_ref[0])
    fw = lax.dot_general(h, w2_ref[...], (((1,), (1,)), ((), ())),
                         preferred_element_type=jnp.float32)
    fw = _prelu(fw + b2_ref[...], a_ref[1])
    wi = lax.bitcast_convert_type(g_ref[...], jnp.int32)
    gs = lax.bitcast_convert_type(wi << 16, jnp.float32)
    gv = lax.bitcast_convert_type(wi & jnp.int32(-65536), jnp.float32)
    fo = gs * fw
    gate = fo[:, :H]
    ge = fo[:, 2 * H:]
    aux = aux_ref[...]
    rinv = 1.0 / aux[:, 3:4]
    ms_ref[...] = fo[:, H:2 * H]
    for k, mv in ((0, mv0_ref), (1, mv1_ref), (2, mv2_ref)):
        mv[...] = (gv[:, k * H:(k + 1) * H] * gate
                   + (aux[:, k:k + 1] * rinv) * ge)


def _edge_stage(rbf_pad, aux, g, w1p, b1, w2, b2, avec, e):
    grid = e // BE
    full = lambda shape: pl.BlockSpec(shape, lambda i: (0, 0))
    blk = lambda w: pl.BlockSpec((BE, w), lambda i: (i, 0))
    o = jax.ShapeDtypeStruct((e, H), jnp.float32)
    return pl.pallas_call(
        _edge_body,
        grid=(grid,),
        in_specs=[
            blk(32), blk(8), blk(3 * H),
            full((H, 32)), full((1, H)), full((3 * H, H)), full((1, 3 * H)),
            pl.BlockSpec(memory_space=pltpu.SMEM),
        ],
        out_specs=[blk(H), blk(H), blk(H), blk(H)],
        out_shape=[o, o, o, o],
    )(rbf_pad, aux, g, w1p, b1.reshape(1, H), w2, b2.reshape(1, 3 * H), avec)


def _scatter_body(nchunk, n, quota,
                  mcs_ref, mc0_ref, mc1_ref, mc2_ref,
                  mvs_ref, mv0_ref, mv1_ref, mv2_ref,
                  dstc_ref, dstv_ref, zeros_ref, out_ref,
                  idx0_ref, idx1_ref, msg0_ref, msg1_ref, w_ref,
                  acc_ref, msem0, msem1, isem0, isem1):
    cid = lax.axis_index("c")
    tid = lax.axis_index("s")
    wid = tid * 2 + cid
    start = wid * quota
    cnt = jnp.clip(nchunk - start, 0, quota)
    zrows = 80
    nzchunk = n // zrows
    ziters = -(-nzchunk // 16)

    mbufs = (msg0_ref, msg1_ref)
    ibufs = (idx0_ref, idx1_ref)
    msems = (msem0, msem1)
    isems = (isem0, isem1)

    passes = ((mcs_ref, mvs_ref), (mc0_ref, mv0_ref),
              (mc1_ref, mv1_ref), (mc2_ref, mv2_ref))
    for p, (msg_che, msg_vdw) in enumerate(passes):
        pltpu.sync_copy(zeros_ref, w_ref)
        for zi in range(ziters):
            c = zi * 16 + tid

            @pl.when(c < nzchunk)
            def _():
                pltpu.sync_copy(w_ref, acc_ref.at[pl.ds(c * zrows, zrows)])
        plsc.subcore_barrier()
        for msg, dref in ((msg_che, dstc_ref), (msg_vdw, dstv_ref)):
            def _start(j, b, msg=msg, dref=dref):
                base = (start + j) * CH
                pltpu.async_copy(msg.at[pl.ds(base, CH)], mbufs[b], msems[b])
                pltpu.async_copy(dref.at[pl.ds(base, CH)], ibufs[b], isems[b])

            for b in (0, 1):
                @pl.when(b < cnt)
                def _(b=b):
                    _start(b, b)

            def body(g, carry, msg=msg, dref=dref, _start=_start):
                for b in (0, 1):
                    j = g * 2 + b

                    @pl.when(j < cnt)
                    def _(j=j, b=b):
                        pltpu.make_async_copy(
                            msg.at[pl.ds(0, CH)], mbufs[b], msems[b]).wait()
                        pltpu.make_async_copy(
                            dref.at[pl.ds(0, CH)], ibufs[b], isems[b]).wait()
                        pltpu.sync_copy(mbufs[b], acc_ref.at[ibufs[b]],
                                        add=True)

                    @pl.when(j + 2 < cnt)
                    def _(j=j, b=b):
                        _start(j + 2, b)
                return carry

            lax.fori_loop(0, quota // 2, body, 0)
        plsc.subcore_barrier()
        obase = (p * 2 + cid) * n
        for zi in range(ziters):
            c = zi * 16 + tid

            @pl.when(c < nzchunk)
            def _():
                pltpu.sync_copy(acc_ref.at[pl.ds(c * zrows, zrows)], w_ref)
                pltpu.sync_copy(w_ref, out_ref.at[pl.ds(obase + c * zrows, zrows)])
        plsc.subcore_barrier()


def _sc_scatter(msgs_che, msgs_vdw, dst_che, dst_vdw, zeros, e, n):
    nchunk = e // CH
    quota = _quota(nchunk)
    mesh = plsc.VectorSubcoreMesh(core_axis_name="c", subcore_axis_name="s")
    kern = pl.kernel(
        functools.partial(_scatter_body, nchunk, n, quota),
        out_type=jax.ShapeDtypeStruct((8 * n, H), jnp.float32),
        mesh=mesh,
        scratch_types=[
            pltpu.VMEM((CH,), jnp.int32),
            pltpu.VMEM((CH,), jnp.int32),
            pltpu.VMEM((CH, H), jnp.float32),
            pltpu.VMEM((CH, H), jnp.float32),
            pltpu.VMEM((80, H), jnp.float32),
            pltpu.VMEM_SHARED((n, H), jnp.float32),
            pltpu.SemaphoreType.DMA,
            pltpu.SemaphoreType.DMA,
            pltpu.SemaphoreType.DMA,
            pltpu.SemaphoreType.DMA,
        ],
    )
    return kern(*msgs_che, *msgs_vdw, dst_che, dst_vdw, zeros)


def _combine_body(ns_ref, nv0_ref, nv1_ref, nv2_ref,
                  ps0_ref, ps1_ref, p00_ref, p01_ref,
                  p10_ref, p11_ref, p20_ref, p21_ref,
                  os_ref, ov0_ref, ov1_ref, ov2_ref):
    os_ref[...] = ns_ref[...] + ps0_ref[...] + ps1_ref[...]
    ov0_ref[...] = nv0_ref[...] + p00_ref[...] + p01_ref[...]
    ov1_ref[...] = nv1_ref[...] + p10_ref[...] + p11_ref[...]
    ov2_ref[...] = nv2_ref[...] + p20_ref[...] + p21_ref[...]


def _combine(node_scalar, nv0, nv1, nv2, part, n):
    grid = n // BN
    nb = n // BN
    blk = pl.BlockSpec((BN, H), lambda i: (i, 0))

    def pblk(p, c):
        off = (p * 2 + c) * nb
        return pl.BlockSpec((BN, H), lambda i, off=off: (off + i, 0))

    o = jax.ShapeDtypeStruct((n, H), jnp.float32)
    return pl.pallas_call(
        _combine_body,
        grid=(grid,),
        in_specs=[blk, blk, blk, blk,
                  pblk(0, 0), pblk(0, 1), pblk(1, 0), pblk(1, 1),
                  pblk(2, 0), pblk(2, 1), pblk(3, 0), pblk(3, 1)],
        out_specs=[blk, blk, blk, blk],
        out_shape=[o, o, o, o],
    )(node_scalar, nv0, nv1, nv2,
      part, part, part, part, part, part, part, part)


def kernel(node_scalar, node_vector, che_edge, che_edge_diff, che_edge_dist,
           che_rbf_dist, vdw_edge, vdw_edge_diff, vdw_edge_dist, vdw_rbf_dist,
           che_s_W1, che_s_b1, che_s_a1, che_s_W2, che_s_b2, che_s_a2,
           che_f_W1, che_f_b1, che_f_a1, che_f_W2, che_f_b2, che_f_a2,
           vdw_s_W1, vdw_s_b1, vdw_s_a1, vdw_s_W2, vdw_s_b2, vdw_s_a2,
           vdw_f_W1, vdw_f_b1, vdw_f_a1, vdw_f_W2, vdw_f_b2, vdw_f_a2):
    n = node_scalar.shape[0]
    e = che_edge.shape[0]
    pad_e = NW * _quota(e // CH) * CH
    pad_g = NW * _quota(e // GCH) * GCH

    a_s = jnp.stack([che_s_a1, che_s_a2, vdw_s_a1, vdw_s_a2])
    tab_nv = node_vector.reshape(n, 3 * H)
    tab_che, tab_vdw = _node_mlp(node_scalar, tab_nv, che_s_W1, che_s_b1,
                                 che_s_W2, che_s_b2, vdw_s_W1, vdw_s_b1,
                                 vdw_s_W2, vdw_s_b2, a_s)

    def _pack(tab):
        t16 = tab.astype(jnp.bfloat16)
        t2 = jnp.stack([t16[:, :3 * H], t16[:, 3 * H:]], axis=-1)
        return lax.bitcast_convert_type(t2, jnp.float32)

    src_che = jnp.pad(che_edge[:, 1], (0, pad_g - e))
    src_vdw = jnp.pad(vdw_edge[:, 1], (0, pad_g - e))
    g_che = _sc_gather(_pack(tab_che), src_che, e)
    g_vdw = _sc_gather(_pack(tab_vdw), src_vdw, e)

    a_fc = jnp.stack([che_f_a1, che_f_a2])
    a_fv = jnp.stack([vdw_f_a1, vdw_f_a2])
    ee = che_rbf_dist.shape[1]
    rbf_che = jnp.pad(che_rbf_dist, ((0, 0), (0, 32 - ee)))
    rbf_vdw = jnp.pad(vdw_rbf_dist, ((0, 0), (0, 32 - ee)))
    w1c = jnp.pad(che_f_W1, ((0, 0), (0, 32 - ee)))
    w1v = jnp.pad(vdw_f_W1, ((0, 0), (0, 32 - ee)))
    aux_che = jnp.pad(
        jnp.concatenate([che_edge_diff, che_edge_dist[:, None]], axis=1),
        ((0, 0), (0, 4)))
    aux_vdw = jnp.pad(
        jnp.concatenate([vdw_edge_diff, vdw_edge_dist[:, None]], axis=1),
        ((0, 0), (0, 4)))
    msgs_che = _edge_stage(rbf_che, aux_che, g_che,
                           w1c, che_f_b1, che_f_W2, che_f_b2, a_fc, e)
    msgs_vdw = _edge_stage(rbf_vdw, aux_vdw, g_vdw,
                           w1v, vdw_f_b1, vdw_f_W2, vdw_f_b2, a_fv, e)

    dst_che = jnp.pad(che_edge[:, 0], (0, pad_e - e))
    dst_vdw = jnp.pad(vdw_edge[:, 0], (0, pad_e - e))
    zeros = jnp.zeros((80, H), jnp.float32)
    part = _sc_scatter(msgs_che, msgs_vdw, dst_che, dst_vdw, zeros, e, n)

    nv0 = node_vector[:, 0, :]
    nv1 = node_vector[:, 1, :]
    nv2 = node_vector[:, 2, :]
    os_, ov0, ov1, ov2 = _combine(node_scalar, nv0, nv1, nv2, part, n)
    return os_, jnp.stack([ov0, ov1, ov2], axis=1)
